# baseline (speedup 1.0000x reference)
import functools

import jax
import jax.numpy as jnp
from jax import lax
from jax.experimental import pallas as pl
from jax.experimental.pallas import tpu as pltpu
from jax.experimental.pallas import tpu_sc as plsc

_IMIN = -2147483648


def _topk_body(K, TQ, base0, xt_ref, xq_ref, fidx_ref):
    b = pl.program_id(0)
    q = pl.program_id(1)
    xt = xt_ref[0]
    xq = xq_ref[0]
    n, c = xt.shape
    inner = lax.dot_general(xq, xt, (((1,), (1,)), ((), ())),
                            preferred_element_type=jnp.float32)
    sq = lax.dot_general(jnp.ones((1, c), jnp.float32), xt * xt,
                         (((1,), (1,)), ((), ())),
                         preferred_element_type=jnp.float32)
    score = 2.0 * inner - sq
    bits = lax.bitcast_convert_type(score, jnp.int32)
    key = jnp.where(bits >= 0, bits, bits ^ jnp.int32(0x7FFFFFFF))
    lane = lax.broadcasted_iota(jnp.int32, (TQ, n), 1)
    key = (key & jnp.int32(-2048)) | (jnp.int32(2047) - lane)
    self_lane = lax.broadcasted_iota(jnp.int32, (TQ, n), 0) + q * TQ
    key = jnp.where(lane == self_lane, jnp.int32(_IMIN), key)
    col = lax.broadcasted_iota(jnp.int32, (TQ, K), 1)
    ids = jnp.zeros((TQ, K), jnp.int32)
    base = b * n + base0
    for j in range(K):
        m = jnp.max(key, axis=1, keepdims=True)
        amj = jnp.int32(2047) - (m & jnp.int32(2047))
        ids = jnp.where(col == j, amj + base, ids)
        key = jnp.where(key == m, jnp.int32(_IMIN), key)
    fidx_ref[0] = ids


def _topk_call(xt, K, TQ, base=0):
    B, N, C = xt.shape
    return pl.pallas_call(
        functools.partial(_topk_body, K, TQ, base),
        grid=(B, N // TQ),
        in_specs=[
            pl.BlockSpec((1, N, C), lambda b, q: (b, 0, 0)),
            pl.BlockSpec((1, TQ, C), lambda b, q: (b, q, 0)),
        ],
        out_specs=pl.BlockSpec((1, TQ, K), lambda b, q: (b, q, 0)),
        out_shape=jax.ShapeDtypeStruct((B, N, K), jnp.int32),
    )(xt, xt)


def _sc_gather(table, idx):
    Rtot = idx.shape[0]
    Cw = table.shape[1]
    dt = table.dtype
    info = plsc.get_sparse_core_info()
    NW = info.num_cores * info.num_subcores
    per_w = Rtot // NW
    J = per_w // 128
    GB = min(J, 8)
    idx3 = idx.reshape(NW, J, 128)
    mesh = plsc.VectorSubcoreMesh(core_axis_name="c", subcore_axis_name="s")

    @functools.partial(
        pl.kernel, mesh=mesh,
        compiler_params=pltpu.CompilerParams(use_tc_tiling_on_sc=False),
        out_type=jax.ShapeDtypeStruct((Rtot // 128, 128, Cw), dt),
        scratch_types=[
            pltpu.VMEM((J, 128), jnp.int32),
            pltpu.VMEM((GB, 128, Cw), dt),
            pltpu.SemaphoreType.DMA,
        ],
    )
    def k(table_hbm, idx_hbm, out_hbm, idx_v, rows_v, sem):
        w = lax.axis_index("s") * info.num_cores + lax.axis_index("c")
        pltpu.sync_copy(idx_hbm.at[w], idx_v)
        def group(g, carry):
            def fire(t, c2):
                pltpu.async_copy(table_hbm.at[idx_v.at[g * GB + t]],
                                 rows_v.at[t], sem)
                return c2
            lax.fori_loop(0, GB, fire, 0)
            def drain(t, c2):
                pltpu.make_async_copy(table_hbm.at[idx_v.at[0]],
                                      rows_v.at[t], sem).wait()
                return c2
            lax.fori_loop(0, GB, drain, 0)
            pltpu.sync_copy(rows_v, out_hbm.at[pl.ds(w * J + g * GB, GB)])
            return carry
        lax.fori_loop(0, J // GB, group, 0)

    return k(table, idx3)


def _conv_body(K, TN, xt_ref, xg_ref, w1d_ref, w1b_ref, w2_ref, w3_ref,
               b1_ref, b2_ref, b3_ref, out_ref):
    R = TN * K
    xt = xt_ref[0]
    xg = xg_ref[0].reshape(R, xt.shape[1]).astype(jnp.bfloat16)
    u = jnp.dot(xt.astype(jnp.bfloat16), w1d_ref[...],
                preferred_element_type=jnp.float32) + b1_ref[...]
    vg = jnp.dot(xg, w1b_ref[...],
                 preferred_element_type=jnp.float32)
    e = u.shape[1]
    h1 = jnp.maximum(vg.reshape(TN, K, e) + u[:, None, :], 0.0).reshape(R, e)
    h2 = jnp.maximum(jnp.dot(h1.astype(jnp.bfloat16), w2_ref[...],
                             preferred_element_type=jnp.float32)
                     + b2_ref[...], 0.0)
    h3 = jnp.dot(h2.astype(jnp.bfloat16), w3_ref[...],
                 preferred_element_type=jnp.float32) + b3_ref[...]
    res = jnp.max(h3.reshape(TN, K, xt.shape[1]), axis=1) + xt
    out_ref[0] = res.T


def _conv_call(xt, xg, w1d, w1b, w2, w3, b1, b2, b3, K, TN):
    B, N, C = xt.shape
    E = w2.shape[0]
    return pl.pallas_call(
        functools.partial(_conv_body, K, TN),
        grid=(B, N // TN),
        in_specs=[
            pl.BlockSpec((1, TN, C), lambda b, q: (b, q, 0)),
            pl.BlockSpec((1, TN * K, C), lambda b, q: (b, q, 0)),
            pl.BlockSpec((C, E), lambda b, q: (0, 0)),
            pl.BlockSpec((C, E), lambda b, q: (0, 0)),
            pl.BlockSpec((E, E), lambda b, q: (0, 0)),
            pl.BlockSpec((E, C), lambda b, q: (0, 0)),
            pl.BlockSpec((1, E), lambda b, q: (0, 0)),
            pl.BlockSpec((1, E), lambda b, q: (0, 0)),
            pl.BlockSpec((1, C), lambda b, q: (0, 0)),
        ],
        out_specs=pl.BlockSpec((1, C, TN), lambda b, q: (b, 0, q)),
        out_shape=jax.ShapeDtypeStruct((B, C, N), jnp.float32),
    )(xt, xg.reshape(B, N * K, C),
      w1d.astype(jnp.bfloat16), w1b.astype(jnp.bfloat16),
      w2.astype(jnp.bfloat16), w3.astype(jnp.bfloat16),
      b1.reshape(1, E), b2.reshape(1, E), b3.reshape(1, C))


def kernel(input, W1, b1, W2, b2, W3, b3):
    x = input
    B, C, N = x.shape
    K = 16
    xt = jnp.transpose(x, (0, 2, 1))
    w1a, w1b = W1[:, :C], W1[:, C:]
    table = xt.reshape(B * N, C)
    fidxs = [_topk_call(xt[h:h + 1], K=K, TQ=512, base=h * N) for h in range(B)]
    H = B // 2
    xgs = [_sc_gather(table, jnp.concatenate(
               [f.reshape(N * K) for f in fidxs[h * H:(h + 1) * H]]))
           for h in range(2)]
    outs = [_conv_call(xt[h * H:(h + 1) * H], xgs[h].reshape(H, N, K, C),
                       (w1a - w1b).T, w1b.T, W2.T, W3.T, b1, b2, b3,
                       K=K, TN=512) for h in range(2)]
    return jnp.concatenate(outs, axis=0)


# speedup vs baseline: 1.0106x; 1.0106x over previous
import functools

import jax
import jax.numpy as jnp
from jax import lax
from jax.experimental import pallas as pl
from jax.experimental.pallas import tpu as pltpu
from jax.experimental.pallas import tpu_sc as plsc

_IMIN = -2147483648


def _topk_body(K, TQ, base0, xt_ref, xq_ref, fidx_ref):
    b = pl.program_id(0)
    q = pl.program_id(1)
    xt = xt_ref[0]
    xq = xq_ref[0]
    n, c = xt.shape
    inner = lax.dot_general(xq, xt, (((1,), (1,)), ((), ())),
                            preferred_element_type=jnp.float32)
    sq = lax.dot_general(jnp.ones((1, c), jnp.float32), xt * xt,
                         (((1,), (1,)), ((), ())),
                         preferred_element_type=jnp.float32)
    score = 2.0 * inner - sq
    bits = lax.bitcast_convert_type(score, jnp.int32)
    key = jnp.where(bits >= 0, bits, bits ^ jnp.int32(0x7FFFFFFF))
    lane = lax.broadcasted_iota(jnp.int32, (TQ, n), 1)
    key = (key & jnp.int32(-2048)) | (jnp.int32(2047) - lane)
    self_lane = lax.broadcasted_iota(jnp.int32, (TQ, n), 0) + q * TQ
    key = jnp.where(lane == self_lane, jnp.int32(_IMIN), key)
    col = lax.broadcasted_iota(jnp.int32, (TQ, K), 1)
    ids = jnp.zeros((TQ, K), jnp.int32)
    base = b * n + base0
    for j in range(K):
        m = jnp.max(key, axis=1, keepdims=True)
        amj = jnp.int32(2047) - (m & jnp.int32(2047))
        ids = jnp.where(col == j, amj + base, ids)
        key = jnp.where(key == m, jnp.int32(_IMIN), key)
    fidx_ref[0] = ids


def _topk_call(xt, K, TQ, base=0):
    B, N, C = xt.shape
    return pl.pallas_call(
        functools.partial(_topk_body, K, TQ, base),
        grid=(B, N // TQ),
        in_specs=[
            pl.BlockSpec((1, N, C), lambda b, q: (b, 0, 0)),
            pl.BlockSpec((1, TQ, C), lambda b, q: (b, q, 0)),
        ],
        out_specs=pl.BlockSpec((1, TQ, K), lambda b, q: (b, q, 0)),
        out_shape=jax.ShapeDtypeStruct((B, N, K), jnp.int32),
    )(xt, xt)


def _sc_gather(table, idx):
    Rtot = idx.shape[0]
    Cw = table.shape[1]
    dt = table.dtype
    info = plsc.get_sparse_core_info()
    NW = info.num_cores * info.num_subcores
    per_w = Rtot // NW
    J = per_w // 128
    GB = min(J, 8)
    idx3 = idx.reshape(NW, J, 128)
    mesh = plsc.VectorSubcoreMesh(core_axis_name="c", subcore_axis_name="s")

    @functools.partial(
        pl.kernel, mesh=mesh,
        compiler_params=pltpu.CompilerParams(use_tc_tiling_on_sc=False),
        out_type=jax.ShapeDtypeStruct((Rtot // 128, 128, Cw), dt),
        scratch_types=[
            pltpu.VMEM((J, 128), jnp.int32),
            pltpu.VMEM((GB, 128, Cw), dt),
            pltpu.SemaphoreType.DMA,
        ],
    )
    def k(table_hbm, idx_hbm, out_hbm, idx_v, rows_v, sem):
        w = lax.axis_index("s") * info.num_cores + lax.axis_index("c")
        pltpu.sync_copy(idx_hbm.at[w], idx_v)
        def group(g, carry):
            def fire(t, c2):
                pltpu.async_copy(table_hbm.at[idx_v.at[g * GB + t]],
                                 rows_v.at[t], sem)
                return c2
            lax.fori_loop(0, GB, fire, 0)
            def drain(t, c2):
                pltpu.make_async_copy(table_hbm.at[idx_v.at[0]],
                                      rows_v.at[t], sem).wait()
                return c2
            lax.fori_loop(0, GB, drain, 0)
            pltpu.sync_copy(rows_v, out_hbm.at[pl.ds(w * J + g * GB, GB)])
            return carry
        lax.fori_loop(0, J // GB, group, 0)

    return k(table, idx3)


def _conv_body(K, TN, xt_ref, xg_ref, w1d_ref, w1b_ref, w2_ref, w3_ref,
               b1_ref, b2_ref, b3_ref, out_ref):
    R = TN * K
    xt = xt_ref[0]
    xg = xg_ref[0].reshape(R, xt.shape[1]).astype(jnp.bfloat16)
    u = jnp.dot(xt.astype(jnp.bfloat16), w1d_ref[...],
                preferred_element_type=jnp.float32) + b1_ref[...]
    vg = jnp.dot(xg, w1b_ref[...],
                 preferred_element_type=jnp.float32)
    e = u.shape[1]
    h1 = jnp.maximum(vg.reshape(TN, K, e) + u[:, None, :], 0.0).reshape(R, e)
    h2 = jnp.maximum(jnp.dot(h1.astype(jnp.bfloat16), w2_ref[...],
                             preferred_element_type=jnp.float32)
                     + b2_ref[...], 0.0)
    h3 = jnp.dot(h2.astype(jnp.bfloat16), w3_ref[...],
                 preferred_element_type=jnp.float32) + b3_ref[...]
    res = jnp.max(h3.reshape(TN, K, xt.shape[1]), axis=1) + xt
    out_ref[0] = res.T


def _conv_call(xt, xg, w1d, w1b, w2, w3, b1, b2, b3, K, TN):
    B, N, C = xt.shape
    E = w2.shape[0]
    return pl.pallas_call(
        functools.partial(_conv_body, K, TN),
        grid=(B, N // TN),
        in_specs=[
            pl.BlockSpec((1, TN, C), lambda b, q: (b, q, 0)),
            pl.BlockSpec((1, TN * K, C), lambda b, q: (b, q, 0)),
            pl.BlockSpec((C, E), lambda b, q: (0, 0)),
            pl.BlockSpec((C, E), lambda b, q: (0, 0)),
            pl.BlockSpec((E, E), lambda b, q: (0, 0)),
            pl.BlockSpec((E, C), lambda b, q: (0, 0)),
            pl.BlockSpec((1, E), lambda b, q: (0, 0)),
            pl.BlockSpec((1, E), lambda b, q: (0, 0)),
            pl.BlockSpec((1, C), lambda b, q: (0, 0)),
        ],
        out_specs=pl.BlockSpec((1, C, TN), lambda b, q: (b, 0, q)),
        out_shape=jax.ShapeDtypeStruct((B, C, N), jnp.float32),
    )(xt, xg.reshape(B, N * K, C),
      w1d.astype(jnp.bfloat16), w1b.astype(jnp.bfloat16),
      w2.astype(jnp.bfloat16), w3.astype(jnp.bfloat16),
      b1.reshape(1, E), b2.reshape(1, E), b3.reshape(1, C))


def kernel(input, W1, b1, W2, b2, W3, b3):
    x = input
    B, C, N = x.shape
    K = 16
    xt = jnp.transpose(x, (0, 2, 1))
    w1a, w1b = W1[:, :C], W1[:, C:]
    table = xt.reshape(B * N, C)
    fidxs = [_topk_call(xt[h:h + 1], K=K, TQ=512, base=h * N) for h in range(B)]
    xgs = [_sc_gather(table, fidxs[h].reshape(N * K)) for h in range(B)]
    outs = [_conv_call(xt[h:h + 1], xgs[h].reshape(1, N, K, C),
                       (w1a - w1b).T, w1b.T, W2.T, W3.T, b1, b2, b3,
                       K=K, TN=512) for h in range(B)]
    return jnp.concatenate(outs, axis=0)
